# Initial kernel scaffold; baseline (speedup 1.0000x reference)
import jax
import jax.numpy as jnp
from jax.experimental import pallas as pl


def kernel(logits):
    raise NotImplementedError("write your pallas kernel here")


# speedup vs baseline: 12.3133x; 12.3133x over previous
import functools

import jax
import jax.numpy as jnp
from jax.experimental import pallas as pl
from jax.experimental.pallas import tpu as pltpu

_ROWS = 128
_N = 32768
_BR = 8
_BISECT_ITERS = 24
_NEWTON_ITERS = 2


def _sparsemax_body(x_ref, o_ref):
    z = x_ref[...]
    m = jnp.max(z, axis=-1, keepdims=True)
    lo = m - 1.0
    hi = m

    def bis(_, carry):
        lo, hi = carry
        mid = 0.5 * (lo + hi)
        f = jnp.sum(jnp.maximum(z - mid, 0.0), axis=-1, keepdims=True)
        ge = f >= 1.0
        return jnp.where(ge, mid, lo), jnp.where(ge, hi, mid)

    lo, hi = jax.lax.fori_loop(0, _BISECT_ITERS, bis, (lo, hi))

    def newton(_, tau):
        sup = z > tau
        k = jnp.sum(sup.astype(jnp.float32), axis=-1, keepdims=True)
        s = jnp.sum(jnp.where(sup, z, 0.0), axis=-1, keepdims=True)
        k = jnp.maximum(k, 1.0)
        return (s - 1.0) / k

    tau = jax.lax.fori_loop(0, _NEWTON_ITERS, newton, lo)
    o_ref[...] = jnp.maximum(z - tau, 0.0)


@jax.jit
def kernel(logits):
    logits = logits.astype(jnp.float32)
    return pl.pallas_call(
        _sparsemax_body,
        grid=(_ROWS // _BR,),
        in_specs=[pl.BlockSpec((_BR, _N), lambda i: (i, 0))],
        out_specs=pl.BlockSpec((_BR, _N), lambda i: (i, 0)),
        out_shape=jax.ShapeDtypeStruct((_ROWS, _N), jnp.float32),
    )(logits)
